# Initial kernel scaffold; baseline (speedup 1.0000x reference)
import jax
import jax.numpy as jnp
from jax import lax
from jax.experimental import pallas as pl
from jax.experimental.pallas import tpu as pltpu
from jax.experimental.pallas import tpu_sc as plsc

NUM_EMB = 1000000
DIM = 64
B_TOTAL = 16384 * 50
CHUNK = 128
NW = 32
ROWS_PER_W = B_TOTAL // (NW * CHUNK)


def _body(idx_hbm, table_hbm, out_hbm, idx_v, buf, gsem):
    nc = 2
    wid = lax.axis_index("s") * nc + lax.axis_index("c")
    row0 = wid * ROWS_PER_W
    pltpu.sync_copy(idx_hbm.at[pl.ds(row0, ROWS_PER_W), :], idx_v)

    @pl.loop(0, ROWS_PER_W)
    def _chunk(j):
        pltpu.async_copy(table_hbm.at[idx_v.at[j]], buf, gsem).wait()
        pltpu.sync_copy(
            buf, out_hbm.at[pl.ds((row0 + j) * CHUNK, CHUNK), :])


@jax.jit
def _gather_sc(idx2d, table):
    mesh = plsc.VectorSubcoreMesh(core_axis_name="c", subcore_axis_name="s")
    return pl.kernel(
        _body,
        out_type=jax.ShapeDtypeStruct((B_TOTAL, DIM), jnp.float32),
        mesh=mesh,
        scratch_types=[
            pltpu.VMEM((ROWS_PER_W, CHUNK), jnp.int32),
            pltpu.VMEM((CHUNK, DIM), jnp.float32),
            pltpu.SemaphoreType.DMA,
        ],
    )(idx2d, table)


def kernel(indices, table):
    idx2d = indices.astype(jnp.int32).reshape(-1, CHUNK)
    out = _gather_sc(idx2d, table)
    return out.reshape(indices.shape[0], indices.shape[1], DIM)


# speedup vs baseline: 1.6835x; 1.6835x over previous
import jax
import jax.numpy as jnp
from jax import lax
from jax.experimental import pallas as pl
from jax.experimental.pallas import tpu as pltpu
from jax.experimental.pallas import tpu_sc as plsc

NUM_EMB = 1000000
DIM = 64
B_TOTAL = 16384 * 50
CHUNK = 128
NW = 32
ROWS_PER_W = B_TOTAL // (NW * CHUNK)


def _body(idx_hbm, table_hbm, out_hbm, idx_v, buf, gsem):
    nc = 2
    wid = lax.axis_index("s") * nc + lax.axis_index("c")
    row0 = wid * ROWS_PER_W
    pltpu.sync_copy(idx_hbm.at[pl.ds(row0, ROWS_PER_W), :], idx_v)

    @pl.loop(0, ROWS_PER_W)
    def _chunk(j):
        pltpu.async_copy(table_hbm.at[idx_v.at[j]], buf, gsem).wait()
        pltpu.sync_copy(
            buf, out_hbm.at[pl.ds((row0 + j) * CHUNK, CHUNK), :])


@jax.jit
def _gather_sc(idx2d, table):
    mesh = plsc.VectorSubcoreMesh(core_axis_name="c", subcore_axis_name="s")
    return pl.kernel(
        _body,
        out_type=jax.ShapeDtypeStruct((B_TOTAL, DIM), jnp.float32),
        mesh=mesh,
        scratch_types=[
            pltpu.VMEM((ROWS_PER_W, CHUNK), jnp.int32),
            pltpu.VMEM((CHUNK, DIM), jnp.float32),
            pltpu.SemaphoreType.DMA,
        ],
        compiler_params=pltpu.CompilerParams(use_tc_tiling_on_sc=False),
    )(idx2d, table)


def kernel(indices, table):
    idx2d = indices.astype(jnp.int32).reshape(-1, CHUNK)
    out = _gather_sc(idx2d, table)
    return out.reshape(indices.shape[0], indices.shape[1], DIM)


# speedup vs baseline: 1.8707x; 1.1112x over previous
import jax
import jax.numpy as jnp
from jax import lax
from jax.experimental import pallas as pl
from jax.experimental.pallas import tpu as pltpu
from jax.experimental.pallas import tpu_sc as plsc

NUM_EMB = 1000000
DIM = 64
B_TOTAL = 16384 * 50
CHUNK = 128
NW = 32
ROWS_PER_W = B_TOTAL // (NW * CHUNK)
NBUF = 8


def _body(idx_hbm, table_hbm, out_hbm, idx_v, *rest):
    bufs = rest[:NBUF]
    gsems = rest[NBUF:2 * NBUF]
    ssems = rest[2 * NBUF:3 * NBUF]
    nc = 2
    wid = lax.axis_index("s") * nc + lax.axis_index("c")
    row0 = wid * ROWS_PER_W
    pltpu.sync_copy(idx_hbm.at[pl.ds(row0, ROWS_PER_W), :], idx_v)

    def out_slice(j):
        return out_hbm.at[pl.ds((row0 + j) * CHUNK, CHUNK), :]

    def fire_gather(j, b):
        pltpu.async_copy(table_hbm.at[idx_v.at[j]], bufs[b], gsems[b])

    def wait_gather(j, b):
        pltpu.make_async_copy(
            table_hbm.at[idx_v.at[j]], bufs[b], gsems[b]).wait()

    for b in range(NBUF):
        fire_gather(b, b)

    @pl.loop(0, ROWS_PER_W - NBUF, step=NBUF)
    def _round(g):
        for b in range(NBUF):
            j = g + b
            wait_gather(j, b)
            pltpu.async_copy(bufs[b], out_slice(j), ssems[b])
        for b in range(NBUF):
            j = g + b
            pltpu.make_async_copy(bufs[b], out_slice(j), ssems[b]).wait()
            fire_gather(j + NBUF, b)

    g_last = ROWS_PER_W - NBUF
    for b in range(NBUF):
        j = g_last + b
        wait_gather(j, b)
        pltpu.async_copy(bufs[b], out_slice(j), ssems[b])
    for b in range(NBUF):
        j = g_last + b
        pltpu.make_async_copy(bufs[b], out_slice(j), ssems[b]).wait()


@jax.jit
def _gather_sc(idx2d, table):
    mesh = plsc.VectorSubcoreMesh(core_axis_name="c", subcore_axis_name="s")
    return pl.kernel(
        _body,
        out_type=jax.ShapeDtypeStruct((B_TOTAL, DIM), jnp.float32),
        mesh=mesh,
        scratch_types=(
            [pltpu.VMEM((ROWS_PER_W, CHUNK), jnp.int32)]
            + [pltpu.VMEM((CHUNK, DIM), jnp.float32) for _ in range(NBUF)]
            + [pltpu.SemaphoreType.DMA for _ in range(2 * NBUF)]
        ),
        compiler_params=pltpu.CompilerParams(use_tc_tiling_on_sc=False),
    )(idx2d, table)


def kernel(indices, table):
    idx2d = indices.astype(jnp.int32).reshape(-1, CHUNK)
    out = _gather_sc(idx2d, table)
    return out.reshape(indices.shape[0], indices.shape[1], DIM)
